# Initial kernel scaffold; baseline (speedup 1.0000x reference)
import jax
import jax.numpy as jnp
from jax.experimental import pallas as pl


def kernel(x, W, b):
    raise NotImplementedError("write your pallas kernel here")


# speedup vs baseline: 5.1162x; 5.1162x over previous
import functools

import jax
import jax.numpy as jnp
from jax.experimental import pallas as pl
from jax.experimental.pallas import tpu as pltpu

_N_TOKENS = 32768
_D_MODEL = 4096
_NUM_EXPERTS = 64
_TOP_K = 8
_BLOCK = 512

_NEG_INF = float("-inf")


def _gating_block(x_ref, w_ref, b_ref, wout_ref, iout_ref):
    x_blk = x_ref[...]
    w = w_ref[...]
    logits = jax.lax.dot_general(
        x_blk, w, (((1,), (1,)), ((), ())),
        preferred_element_type=jnp.float32,
    ) + b_ref[...]

    rows = logits.shape[0]
    lane = jax.lax.broadcasted_iota(jnp.int32, (rows, _NUM_EXPERTS), 1)

    work = logits
    sel = jnp.zeros_like(logits, dtype=jnp.bool_)
    row_max = None
    idx_cols = []
    for j in range(_TOP_K):
        m = jnp.max(work, axis=1, keepdims=True)
        if j == 0:
            row_max = m
        idx = jnp.min(
            jnp.where(work == m, lane, jnp.int32(_NUM_EXPERTS)),
            axis=1, keepdims=True,
        )
        idx_cols.append(idx)
        hit = lane == idx
        sel = jnp.logical_or(sel, hit)
        work = jnp.where(hit, _NEG_INF, work)

    ew = jnp.where(sel, jnp.exp(logits - row_max), 0.0)
    denom = jnp.sum(ew, axis=1, keepdims=True)
    wout_ref[...] = ew / denom
    iout_ref[...] = jnp.concatenate(idx_cols, axis=1)


@jax.jit
def kernel(x, W, b):
    n_tokens = x.shape[0]
    grid = (n_tokens // _BLOCK,)
    b2d = b.reshape(1, _NUM_EXPERTS)
    weights, indices = pl.pallas_call(
        _gating_block,
        grid=grid,
        in_specs=[
            pl.BlockSpec((_BLOCK, _D_MODEL), lambda i: (i, 0)),
            pl.BlockSpec((_NUM_EXPERTS, _D_MODEL), lambda i: (0, 0)),
            pl.BlockSpec((1, _NUM_EXPERTS), lambda i: (0, 0)),
        ],
        out_specs=[
            pl.BlockSpec((_BLOCK, _NUM_EXPERTS), lambda i: (i, 0)),
            pl.BlockSpec((_BLOCK, _TOP_K), lambda i: (i, 0)),
        ],
        out_shape=[
            jax.ShapeDtypeStruct((n_tokens, _NUM_EXPERTS), jnp.float32),
            jax.ShapeDtypeStruct((n_tokens, _TOP_K), jnp.int32),
        ],
        compiler_params=pltpu.CompilerParams(
            dimension_semantics=("parallel",),
        ),
    )(x, W, b2d)
    return weights, indices
